# baseline (speedup 1.0000x reference)
import functools

import jax
import jax.numpy as jnp
from jax import lax
from jax.experimental import pallas as pl
from jax.experimental.pallas import tpu as pltpu
from jax.experimental.pallas import tpu_sc as plsc

N = 10000
E = 320000
NC = 2
NS = 16
NW = NC * NS
EW = E // NW
RPT = N // NS


@functools.lru_cache(maxsize=None)
def _make_sc_agg(D: int):
    mesh = plsc.VectorSubcoreMesh(core_axis_name="c", subcore_axis_name="s")
    CH = 40 if D == 128 else 80
    NCH = EW // CH
    NB = 6 if D == 128 else 8
    scratch = [
        pltpu.VMEM((EW,), jnp.int32),
        pltpu.VMEM((EW,), jnp.int32),
        [pltpu.VMEM((CH, D), jnp.float32) for _ in range(NB)],
        pltpu.VMEM_SHARED((N, D), jnp.float32),
        [pltpu.SemaphoreType.DMA for _ in range(NB)],
        pltpu.SemaphoreType.DMA,
    ]

    @functools.partial(
        pl.kernel,
        mesh=mesh,
        out_type=jax.ShapeDtypeStruct((NC, N, D), jnp.float32),
        scratch_types=scratch,
        compiler_params=pltpu.CompilerParams(use_tc_tiling_on_sc=False),
    )
    def k(x_hbm, ei_hbm, out_hbm, idxs, idxd, bufs, acc, sems, semi):
        c = lax.axis_index("c")
        s = lax.axis_index("s")
        wid = s * NC + c
        ebase = wid * EW
        pltpu.async_copy(ei_hbm.at[pl.ds(ebase, EW)], idxs, semi)
        pltpu.async_copy(ei_hbm.at[pl.ds(E + ebase, EW)], idxd, semi)
        pltpu.sync_copy(x_hbm.at[pl.ds(s * RPT, RPT)], acc.at[pl.ds(s * RPT, RPT)])
        pltpu.make_async_copy(ei_hbm.at[pl.ds(ebase, EW)], idxs, semi).wait()
        pltpu.make_async_copy(ei_hbm.at[pl.ds(E + ebase, EW)], idxd, semi).wait()
        plsc.subcore_barrier()

        def gstart(j, rb, sem):
            pltpu.async_copy(x_hbm.at[idxs.at[pl.ds(j * CH, CH)]], rb, sem)

        def gwait(j, rb, sem):
            pltpu.make_async_copy(x_hbm.at[idxs.at[pl.ds(j * CH, CH)]], rb,
                                  sem).wait()

        def sstart(j, rb, sem):
            pltpu.async_copy(rb, acc.at[idxd.at[pl.ds(j * CH, CH)]], sem,
                             add=True)

        def swait(j, rb, sem):
            pltpu.make_async_copy(rb, acc.at[idxd.at[pl.ds(j * CH, CH)]],
                                  sem).wait()

        G = NB // 2
        M = NCH // NB
        TAIL = NCH - M * NB
        assert G <= TAIL and TAIL - G <= G
        A, B = bufs[:G], bufs[G:]
        sA, sB = sems[:G], sems[G:]
        for t in range(G):
            gstart(t, A[t], sA[t])
        for t in range(G):
            gwait(t, A[t], sA[t])
            sstart(t, A[t], sA[t])
        for t in range(G):
            gstart(G + t, B[t], sB[t])
        for t in range(G):
            gwait(G + t, B[t], sB[t])
            sstart(G + t, B[t], sB[t])
        for t in range(G):
            swait(t, A[t], sA[t])
            gstart(NB + t, A[t], sA[t])

        def period(m, carry):
            jb = m * NB
            for t in range(G):
                gwait(jb + t, A[t], sA[t])
                sstart(jb + t, A[t], sA[t])
            for t in range(G):
                swait(jb + G + t - NB, B[t], sB[t])
                gstart(jb + G + t, B[t], sB[t])
            for t in range(G):
                gwait(jb + G + t, B[t], sB[t])
                sstart(jb + G + t, B[t], sB[t])
            for t in range(G):
                swait(jb + t, A[t], sA[t])
                gstart(jb + NB + t, A[t], sA[t])
            return carry

        lax.fori_loop(1, M, period, 0)
        for t in range(G):
            gwait(M * NB + t, A[t], sA[t])
            sstart(M * NB + t, A[t], sA[t])
        for t in range(TAIL - G):
            swait((M - 1) * NB + G + t, B[t], sB[t])
            gstart(M * NB + G + t, B[t], sB[t])
        for t in range(TAIL - G):
            gwait(M * NB + G + t, B[t], sB[t])
            sstart(M * NB + G + t, B[t], sB[t])
        for t in range(TAIL - G, G):
            swait((M - 1) * NB + G + t, B[t], sB[t])
        for t in range(G):
            swait(M * NB + t, A[t], sA[t])
        for t in range(TAIL - G):
            swait(M * NB + G + t, B[t], sB[t])
        plsc.subcore_barrier()
        pltpu.sync_copy(acc.at[pl.ds(s * RPT, RPT)],
                        out_hbm.at[c, pl.ds(s * RPT, RPT)])

    return k


BN = 2000


N4 = N // 4


def _tc1_body(eps_ref, x_ref, agg_ref, W1a_ref, b1a_ref, W1b_ref, b1b_ref,
              W2a_ref, u2_ref):
    z1 = agg_ref[0] + agg_ref[1] + (eps_ref[0, 0] - 1.0) * x_ref[...]
    t = jax.nn.relu(jnp.dot(z1, W1a_ref[...], preferred_element_type=jnp.float32, precision=jax.lax.Precision.HIGHEST)
                    + b1a_ref[...])
    h1 = jax.nn.relu(jnp.dot(t, W1b_ref[...], preferred_element_type=jnp.float32, precision=jax.lax.Precision.HIGHEST)
                     + b1b_ref[...])
    u2_ref[...] = jnp.dot(h1, W2a_ref[...], preferred_element_type=jnp.float32, precision=jax.lax.Precision.HIGHEST)


def _tc1(eps1, x, agg1, W1a, b1a, W1b, b1b, W2a):
    grid = (N // BN,)
    return pl.pallas_call(
        _tc1_body,
        grid=grid,
        in_specs=[
            pl.BlockSpec(memory_space=pltpu.SMEM),
            pl.BlockSpec((BN, 128), lambda i: (i, 0)),
            pl.BlockSpec((NC, BN, 128), lambda i: (0, i, 0)),
            pl.BlockSpec((128, 128), lambda i: (0, 0)),
            pl.BlockSpec((1, 128), lambda i: (0, 0)),
            pl.BlockSpec((128, 128), lambda i: (0, 0)),
            pl.BlockSpec((1, 128), lambda i: (0, 0)),
            pl.BlockSpec((128, 32), lambda i: (0, 0)),
        ],
        out_specs=pl.BlockSpec((BN, 32), lambda i: (i, 0)),
        out_shape=jax.ShapeDtypeStruct((N, 32), jnp.float32),
    )(eps1, x, agg1, W1a, b1a, W1b, b1b, W2a)


def _tc2_body(eps_ref, u2_ref, agg_ref, b2a_ref, W2b_ref, b2b_ref, Wh_ref,
              bh_ref, out_ref):
    z = agg_ref[0] + agg_ref[1] + (eps_ref[0, 0] - 1.0) * u2_ref[...] + b2a_ref[...]
    t = jax.nn.relu(z)
    h2 = jax.nn.relu(jnp.dot(t, W2b_ref[...], preferred_element_type=jnp.float32, precision=jax.lax.Precision.HIGHEST)
                     + b2b_ref[...])
    out_ref[...] = jnp.dot(h2, Wh_ref[...], preferred_element_type=jnp.float32, precision=jax.lax.Precision.HIGHEST) \
        + bh_ref[...]


def _tc2(eps2, u2p, agg2p, b2a4, W2b_bd, b2b4, Wh_bd, bh4):
    grid = (1,)
    return pl.pallas_call(
        _tc2_body,
        grid=grid,
        in_specs=[
            pl.BlockSpec(memory_space=pltpu.SMEM),
            pl.BlockSpec((N4, 128), lambda i: (0, 0)),
            pl.BlockSpec((NC, N4, 128), lambda i: (0, 0, 0)),
            pl.BlockSpec((1, 128), lambda i: (0, 0)),
            pl.BlockSpec((128, 128), lambda i: (0, 0)),
            pl.BlockSpec((1, 128), lambda i: (0, 0)),
            pl.BlockSpec((128, 16), lambda i: (0, 0)),
            pl.BlockSpec((1, 16), lambda i: (0, 0)),
        ],
        out_specs=pl.BlockSpec((N4, 16), lambda i: (0, 0)),
        out_shape=jax.ShapeDtypeStruct((N4, 16), jnp.float32),
    )(eps2, u2p, agg2p, b2a4, W2b_bd, b2b4, Wh_bd, bh4)


def kernel(x, edge_index, eps1, W1a, b1a, W1b, b1b, eps2, W2a, b2a, W2b, b2b,
           Wh, bh):
    ei = jnp.reshape(edge_index, (2 * E,))
    eps1_s = jnp.reshape(eps1, (1, 1))
    eps2_s = jnp.reshape(eps2, (1, 1))
    agg1 = _make_sc_agg(128)(x, ei)
    u2 = _tc1(eps1_s, x, agg1,
              W1a, jnp.reshape(b1a, (1, 128)),
              W1b, jnp.reshape(b1b, (1, 128)), W2a)
    u2p = jnp.reshape(u2, (N4, 128))
    agg2 = _make_sc_agg(32)(u2, ei)
    agg2p = jnp.reshape(agg2, (NC, N4, 128))
    W2b_bd = jnp.concatenate(
        [jnp.pad(W2b, ((32 * j, 96 - 32 * j), (0, 0))) for j in range(4)],
        axis=1)
    Wh_bd = jnp.concatenate(
        [jnp.pad(Wh, ((32 * j, 96 - 32 * j), (0, 0))) for j in range(4)],
        axis=1)
    outp = _tc2(eps2_s, u2p, agg2p,
                jnp.reshape(jnp.tile(b2a, 4), (1, 128)),
                W2b_bd,
                jnp.reshape(jnp.tile(b2b, 4), (1, 128)),
                Wh_bd,
                jnp.reshape(jnp.tile(bh, 4), (1, 16)))
    return jnp.reshape(outp, (N, 4))


# speedup vs baseline: 1.0412x; 1.0412x over previous
import functools

import jax
import jax.numpy as jnp
from jax import lax
from jax.experimental import pallas as pl
from jax.experimental.pallas import tpu as pltpu
from jax.experimental.pallas import tpu_sc as plsc

N = 10000
E = 320000
NC = 2
NS = 16
NW = NC * NS
EW = E // NW
RPT = N // NS


@functools.lru_cache(maxsize=None)
def _make_sc_agg(D: int):
    mesh = plsc.VectorSubcoreMesh(core_axis_name="c", subcore_axis_name="s")
    CH = 40 if D == 128 else 80
    NCH = EW // CH
    NB = 6 if D == 128 else 8
    via_spmem = D <= 32
    scratch = [
        pltpu.VMEM((EW,), jnp.int32),
        pltpu.VMEM((EW,), jnp.int32),
        [pltpu.VMEM((CH, D), jnp.float32) for _ in range(NB)],
        pltpu.VMEM_SHARED((N, D), jnp.float32),
        [pltpu.SemaphoreType.DMA for _ in range(NB)],
        pltpu.SemaphoreType.DMA,
    ]
    if via_spmem:
        scratch.append(pltpu.VMEM_SHARED((N, D), jnp.float32))

    @functools.partial(
        pl.kernel,
        mesh=mesh,
        out_type=jax.ShapeDtypeStruct((NC, N, D), jnp.float32),
        scratch_types=scratch,
        compiler_params=pltpu.CompilerParams(use_tc_tiling_on_sc=False),
    )
    def k(x_hbm, ei_hbm, out_hbm, idxs, idxd, bufs, acc, sems, semi,
          *maybe_xsp):
        c = lax.axis_index("c")
        s = lax.axis_index("s")
        wid = s * NC + c
        ebase = wid * EW
        gsrc = maybe_xsp[0] if via_spmem else x_hbm
        pltpu.async_copy(ei_hbm.at[pl.ds(ebase, EW)], idxs, semi)
        pltpu.async_copy(ei_hbm.at[pl.ds(E + ebase, EW)], idxd, semi)
        pltpu.sync_copy(x_hbm.at[pl.ds(s * RPT, RPT)], acc.at[pl.ds(s * RPT, RPT)])
        if via_spmem:
            pltpu.sync_copy(x_hbm.at[pl.ds(s * RPT, RPT)],
                            gsrc.at[pl.ds(s * RPT, RPT)])
        pltpu.make_async_copy(ei_hbm.at[pl.ds(ebase, EW)], idxs, semi).wait()
        pltpu.make_async_copy(ei_hbm.at[pl.ds(E + ebase, EW)], idxd, semi).wait()
        plsc.subcore_barrier()

        def gstart(j, rb, sem):
            pltpu.async_copy(gsrc.at[idxs.at[pl.ds(j * CH, CH)]], rb, sem)

        def gwait(j, rb, sem):
            pltpu.make_async_copy(gsrc.at[idxs.at[pl.ds(j * CH, CH)]], rb,
                                  sem).wait()

        def sstart(j, rb, sem):
            pltpu.async_copy(rb, acc.at[idxd.at[pl.ds(j * CH, CH)]], sem,
                             add=True)

        def swait(j, rb, sem):
            pltpu.make_async_copy(rb, acc.at[idxd.at[pl.ds(j * CH, CH)]],
                                  sem).wait()

        NRND = NCH // NB
        TAIL = NCH % NB
        for t in range(NB):
            gstart(t, bufs[t], sems[t])

        def rnd(q, carry):
            jb = q * NB
            for t in range(NB):
                gwait(jb + t, bufs[t], sems[t])
                sstart(jb + t, bufs[t], sems[t])
            for t in range(NB):
                swait(jb + t, bufs[t], sems[t])
                gstart(jb + NB + t, bufs[t], sems[t])
            return carry

        lax.fori_loop(0, NRND - 1, rnd, 0)
        jb = (NRND - 1) * NB
        for t in range(NB):
            gwait(jb + t, bufs[t], sems[t])
            sstart(jb + t, bufs[t], sems[t])
        for t in range(TAIL):
            swait(jb + t, bufs[t], sems[t])
            gstart(jb + NB + t, bufs[t], sems[t])
        for t in range(TAIL):
            gwait(jb + NB + t, bufs[t], sems[t])
            sstart(jb + NB + t, bufs[t], sems[t])
        for t in range(TAIL):
            swait(jb + NB + t, bufs[t], sems[t])
        for t in range(TAIL, NB):
            swait(jb + t, bufs[t], sems[t])
        plsc.subcore_barrier()
        pltpu.sync_copy(acc.at[pl.ds(s * RPT, RPT)],
                        out_hbm.at[c, pl.ds(s * RPT, RPT)])

    return k


BN = 2000


N4 = N // 4


def _tc1_body(eps_ref, x_ref, agg_ref, W1a_ref, b1a_ref, W1b_ref, b1b_ref,
              W2a_ref, u2_ref):
    z1 = agg_ref[0] + agg_ref[1] + (eps_ref[0, 0] - 1.0) * x_ref[...]
    t = jax.nn.relu(jnp.dot(z1, W1a_ref[...], preferred_element_type=jnp.float32, precision=jax.lax.Precision.HIGHEST)
                    + b1a_ref[...])
    h1 = jax.nn.relu(jnp.dot(t, W1b_ref[...], preferred_element_type=jnp.float32, precision=jax.lax.Precision.HIGHEST)
                     + b1b_ref[...])
    u2_ref[...] = jnp.dot(h1, W2a_ref[...], preferred_element_type=jnp.float32, precision=jax.lax.Precision.HIGHEST)


def _tc1(eps1, x, agg1, W1a, b1a, W1b, b1b, W2a):
    grid = (N // BN,)
    return pl.pallas_call(
        _tc1_body,
        grid=grid,
        in_specs=[
            pl.BlockSpec(memory_space=pltpu.SMEM),
            pl.BlockSpec((BN, 128), lambda i: (i, 0)),
            pl.BlockSpec((NC, BN, 128), lambda i: (0, i, 0)),
            pl.BlockSpec((128, 128), lambda i: (0, 0)),
            pl.BlockSpec((1, 128), lambda i: (0, 0)),
            pl.BlockSpec((128, 128), lambda i: (0, 0)),
            pl.BlockSpec((1, 128), lambda i: (0, 0)),
            pl.BlockSpec((128, 32), lambda i: (0, 0)),
        ],
        out_specs=pl.BlockSpec((BN, 32), lambda i: (i, 0)),
        out_shape=jax.ShapeDtypeStruct((N, 32), jnp.float32),
    )(eps1, x, agg1, W1a, b1a, W1b, b1b, W2a)


def _tc2_body(eps_ref, u2_ref, agg_ref, b2a_ref, W2b_ref, b2b_ref, Wh_ref,
              bh_ref, out_ref):
    z = agg_ref[0] + agg_ref[1] + (eps_ref[0, 0] - 1.0) * u2_ref[...] + b2a_ref[...]
    t = jax.nn.relu(z)
    h2 = jax.nn.relu(jnp.dot(t, W2b_ref[...], preferred_element_type=jnp.float32, precision=jax.lax.Precision.HIGHEST)
                     + b2b_ref[...])
    out_ref[...] = jnp.dot(h2, Wh_ref[...], preferred_element_type=jnp.float32, precision=jax.lax.Precision.HIGHEST) \
        + bh_ref[...]


def _tc2(eps2, u2p, agg2p, b2a4, W2b_bd, b2b4, Wh_bd, bh4):
    grid = (1,)
    return pl.pallas_call(
        _tc2_body,
        grid=grid,
        in_specs=[
            pl.BlockSpec(memory_space=pltpu.SMEM),
            pl.BlockSpec((N4, 128), lambda i: (0, 0)),
            pl.BlockSpec((NC, N4, 128), lambda i: (0, 0, 0)),
            pl.BlockSpec((1, 128), lambda i: (0, 0)),
            pl.BlockSpec((128, 128), lambda i: (0, 0)),
            pl.BlockSpec((1, 128), lambda i: (0, 0)),
            pl.BlockSpec((128, 16), lambda i: (0, 0)),
            pl.BlockSpec((1, 16), lambda i: (0, 0)),
        ],
        out_specs=pl.BlockSpec((N4, 16), lambda i: (0, 0)),
        out_shape=jax.ShapeDtypeStruct((N4, 16), jnp.float32),
    )(eps2, u2p, agg2p, b2a4, W2b_bd, b2b4, Wh_bd, bh4)


def kernel(x, edge_index, eps1, W1a, b1a, W1b, b1b, eps2, W2a, b2a, W2b, b2b,
           Wh, bh):
    ei = jnp.reshape(edge_index, (2 * E,))
    eps1_s = jnp.reshape(eps1, (1, 1))
    eps2_s = jnp.reshape(eps2, (1, 1))
    agg1 = _make_sc_agg(128)(x, ei)
    u2 = _tc1(eps1_s, x, agg1,
              W1a, jnp.reshape(b1a, (1, 128)),
              W1b, jnp.reshape(b1b, (1, 128)), W2a)
    u2p = jnp.reshape(u2, (N4, 128))
    agg2 = _make_sc_agg(32)(u2, ei)
    agg2p = jnp.reshape(agg2, (NC, N4, 128))
    W2b_bd = jnp.concatenate(
        [jnp.pad(W2b, ((32 * j, 96 - 32 * j), (0, 0))) for j in range(4)],
        axis=1)
    Wh_bd = jnp.concatenate(
        [jnp.pad(Wh, ((32 * j, 96 - 32 * j), (0, 0))) for j in range(4)],
        axis=1)
    outp = _tc2(eps2_s, u2p, agg2p,
                jnp.reshape(jnp.tile(b2a, 4), (1, 128)),
                W2b_bd,
                jnp.reshape(jnp.tile(b2b, 4), (1, 128)),
                Wh_bd,
                jnp.reshape(jnp.tile(bh, 4), (1, 16)))
    return jnp.reshape(outp, (N, 4))


# speedup vs baseline: 1.1055x; 1.0618x over previous
import functools

import jax
import jax.numpy as jnp
from jax import lax
from jax.experimental import pallas as pl
from jax.experimental.pallas import tpu as pltpu
from jax.experimental.pallas import tpu_sc as plsc

N = 10000
E = 320000
NC = 2
NS = 16
NW = NC * NS
EW = E // NW
RPT = N // NS


@functools.lru_cache(maxsize=None)
def _make_sc_agg(D: int):
    mesh = plsc.VectorSubcoreMesh(core_axis_name="c", subcore_axis_name="s")
    CH = 40 if D == 128 else 80
    NCH = EW // CH
    NB = 6 if D == 128 else 8
    via_spmem = False
    scratch = [
        pltpu.VMEM((EW,), jnp.int32),
        pltpu.VMEM((EW,), jnp.int32),
        [pltpu.VMEM((CH, D), jnp.float32) for _ in range(NB)],
        pltpu.VMEM_SHARED((N, D), jnp.float32),
        [pltpu.SemaphoreType.DMA for _ in range(NB)],
        pltpu.SemaphoreType.DMA,
    ]
    if via_spmem:
        scratch.append(pltpu.VMEM_SHARED((N, D), jnp.float32))

    @functools.partial(
        pl.kernel,
        mesh=mesh,
        out_type=jax.ShapeDtypeStruct((NC, N, D), jnp.float32),
        scratch_types=scratch,
        compiler_params=pltpu.CompilerParams(use_tc_tiling_on_sc=False),
    )
    def k(x_hbm, ei_hbm, out_hbm, idxs, idxd, bufs, acc, sems, semi,
          *maybe_xsp):
        c = lax.axis_index("c")
        s = lax.axis_index("s")
        wid = s * NC + c
        ebase = wid * EW
        gsrc = maybe_xsp[0] if via_spmem else x_hbm
        pltpu.async_copy(ei_hbm.at[pl.ds(ebase, EW)], idxs, semi)
        pltpu.async_copy(ei_hbm.at[pl.ds(E + ebase, EW)], idxd, semi)
        pltpu.sync_copy(x_hbm.at[pl.ds(s * RPT, RPT)], acc.at[pl.ds(s * RPT, RPT)])
        if via_spmem:
            pltpu.sync_copy(x_hbm.at[pl.ds(s * RPT, RPT)],
                            gsrc.at[pl.ds(s * RPT, RPT)])
        pltpu.make_async_copy(ei_hbm.at[pl.ds(ebase, EW)], idxs, semi).wait()
        pltpu.make_async_copy(ei_hbm.at[pl.ds(E + ebase, EW)], idxd, semi).wait()
        plsc.subcore_barrier()

        def gstart(j, rb, sem):
            pltpu.async_copy(gsrc.at[idxs.at[pl.ds(j * CH, CH)]], rb, sem)

        def gwait(j, rb, sem):
            pltpu.make_async_copy(gsrc.at[idxs.at[pl.ds(j * CH, CH)]], rb,
                                  sem).wait()

        def sstart(j, rb, sem):
            pltpu.async_copy(rb, acc.at[idxd.at[pl.ds(j * CH, CH)]], sem,
                             add=True)

        def swait(j, rb, sem):
            pltpu.make_async_copy(rb, acc.at[idxd.at[pl.ds(j * CH, CH)]],
                                  sem).wait()

        NRND = NCH // NB
        TAIL = NCH % NB
        for t in range(NB):
            gstart(t, bufs[t], sems[t])

        def rnd(q, carry):
            jb = q * NB
            for t in range(NB):
                gwait(jb + t, bufs[t], sems[t])
                sstart(jb + t, bufs[t], sems[t])
            for t in range(NB):
                swait(jb + t, bufs[t], sems[t])
                gstart(jb + NB + t, bufs[t], sems[t])
            return carry

        lax.fori_loop(0, NRND - 1, rnd, 0)
        jb = (NRND - 1) * NB
        for t in range(NB):
            gwait(jb + t, bufs[t], sems[t])
            sstart(jb + t, bufs[t], sems[t])
        for t in range(TAIL):
            swait(jb + t, bufs[t], sems[t])
            gstart(jb + NB + t, bufs[t], sems[t])
        for t in range(TAIL):
            gwait(jb + NB + t, bufs[t], sems[t])
            sstart(jb + NB + t, bufs[t], sems[t])
        for t in range(TAIL):
            swait(jb + NB + t, bufs[t], sems[t])
        for t in range(TAIL, NB):
            swait(jb + t, bufs[t], sems[t])
        plsc.subcore_barrier()
        pltpu.sync_copy(acc.at[pl.ds(s * RPT, RPT)],
                        out_hbm.at[c, pl.ds(s * RPT, RPT)])

    return k


BN = 2000


N4 = N // 4


def _tc1_body(eps_ref, x_ref, agg_ref, W1a_ref, b1a_ref, W1b_ref, b1b_ref,
              W2a_ref, u2_ref):
    z1 = agg_ref[0] + agg_ref[1] + (eps_ref[0, 0] - 1.0) * x_ref[...]
    t = jax.nn.relu(jnp.dot(z1, W1a_ref[...], preferred_element_type=jnp.float32, precision=jax.lax.Precision.HIGHEST)
                    + b1a_ref[...])
    h1 = jax.nn.relu(jnp.dot(t, W1b_ref[...], preferred_element_type=jnp.float32, precision=jax.lax.Precision.HIGHEST)
                     + b1b_ref[...])
    u2_ref[...] = jnp.dot(h1, W2a_ref[...], preferred_element_type=jnp.float32, precision=jax.lax.Precision.HIGHEST)


def _tc1(eps1, x, agg1, W1a, b1a, W1b, b1b, W2a):
    grid = (N // BN,)
    return pl.pallas_call(
        _tc1_body,
        grid=grid,
        in_specs=[
            pl.BlockSpec(memory_space=pltpu.SMEM),
            pl.BlockSpec((BN, 128), lambda i: (i, 0)),
            pl.BlockSpec((NC, BN, 128), lambda i: (0, i, 0)),
            pl.BlockSpec((128, 128), lambda i: (0, 0)),
            pl.BlockSpec((1, 128), lambda i: (0, 0)),
            pl.BlockSpec((128, 128), lambda i: (0, 0)),
            pl.BlockSpec((1, 128), lambda i: (0, 0)),
            pl.BlockSpec((128, 32), lambda i: (0, 0)),
        ],
        out_specs=pl.BlockSpec((BN, 32), lambda i: (i, 0)),
        out_shape=jax.ShapeDtypeStruct((N, 32), jnp.float32),
    )(eps1, x, agg1, W1a, b1a, W1b, b1b, W2a)


def _tc2_body(eps_ref, u2_ref, agg_ref, b2a_ref, W2b_ref, b2b_ref, Wh_ref,
              bh_ref, out_ref):
    z = agg_ref[0] + agg_ref[1] + (eps_ref[0, 0] - 1.0) * u2_ref[...] + b2a_ref[...]
    t = jax.nn.relu(z)
    h2 = jax.nn.relu(jnp.dot(t, W2b_ref[...], preferred_element_type=jnp.float32, precision=jax.lax.Precision.HIGHEST)
                     + b2b_ref[...])
    out_ref[...] = jnp.dot(h2, Wh_ref[...], preferred_element_type=jnp.float32, precision=jax.lax.Precision.HIGHEST) \
        + bh_ref[...]


def _tc2(eps2, u2p, agg2p, b2a4, W2b_bd, b2b4, Wh_bd, bh4):
    grid = (1,)
    return pl.pallas_call(
        _tc2_body,
        grid=grid,
        in_specs=[
            pl.BlockSpec(memory_space=pltpu.SMEM),
            pl.BlockSpec((N4, 128), lambda i: (0, 0)),
            pl.BlockSpec((NC, N4, 128), lambda i: (0, 0, 0)),
            pl.BlockSpec((1, 128), lambda i: (0, 0)),
            pl.BlockSpec((128, 128), lambda i: (0, 0)),
            pl.BlockSpec((1, 128), lambda i: (0, 0)),
            pl.BlockSpec((128, 16), lambda i: (0, 0)),
            pl.BlockSpec((1, 16), lambda i: (0, 0)),
        ],
        out_specs=pl.BlockSpec((N4, 16), lambda i: (0, 0)),
        out_shape=jax.ShapeDtypeStruct((N4, 16), jnp.float32),
    )(eps2, u2p, agg2p, b2a4, W2b_bd, b2b4, Wh_bd, bh4)


def kernel(x, edge_index, eps1, W1a, b1a, W1b, b1b, eps2, W2a, b2a, W2b, b2b,
           Wh, bh):
    ei = jnp.reshape(edge_index, (2 * E,))
    eps1_s = jnp.reshape(eps1, (1, 1))
    eps2_s = jnp.reshape(eps2, (1, 1))
    agg1 = _make_sc_agg(128)(x, ei)
    u2 = _tc1(eps1_s, x, agg1,
              W1a, jnp.reshape(b1a, (1, 128)),
              W1b, jnp.reshape(b1b, (1, 128)), W2a)
    u2p = jnp.reshape(u2, (N4, 128))
    agg2 = _make_sc_agg(32)(u2, ei)
    agg2p = jnp.reshape(agg2, (NC, N4, 128))
    W2b_bd = jnp.concatenate(
        [jnp.pad(W2b, ((32 * j, 96 - 32 * j), (0, 0))) for j in range(4)],
        axis=1)
    Wh_bd = jnp.concatenate(
        [jnp.pad(Wh, ((32 * j, 96 - 32 * j), (0, 0))) for j in range(4)],
        axis=1)
    outp = _tc2(eps2_s, u2p, agg2p,
                jnp.reshape(jnp.tile(b2a, 4), (1, 128)),
                W2b_bd,
                jnp.reshape(jnp.tile(b2b, 4), (1, 128)),
                Wh_bd,
                jnp.reshape(jnp.tile(bh, 4), (1, 16)))
    return jnp.reshape(outp, (N, 4))
